# baseline (speedup 1.0000x reference)
import functools

import jax
import jax.numpy as jnp
from jax import lax
from jax.experimental import pallas as pl
from jax.experimental.pallas import tpu as pltpu
from jax.experimental.pallas import tpu_sc as plsc

VOCAB, EMBED_DIM, HIDDEN_DIM = 1000000, 64, 128
B, L = 16384, 50
LP = 56
NC, NS = 2, 16
NW = NC * NS
RPW = B // NW
CB = 1
CHUNK = CB * LP
NCH = RPW // CB
NBUF = 8
NLG = EMBED_DIM // 16


def _sc_pool_tower(table128, idx_flat):
    mesh = plsc.VectorSubcoreMesh(core_axis_name="c", subcore_axis_name="s")

    @functools.partial(
        pl.kernel,
        out_type=jax.ShapeDtypeStruct((B * EMBED_DIM,), jnp.float32),
        mesh=mesh,
        scratch_types=[
            pltpu.VMEM((RPW * LP,), jnp.int32),
            pltpu.VMEM((NBUF, CHUNK, 128), jnp.float32),
            pltpu.VMEM((RPW * EMBED_DIM,), jnp.float32),
            pltpu.SemaphoreType.DMA((NBUF,)),
        ],
    )
    def k(table, idx_hbm, out_hbm, idx_v, buf_v, out_v, sems):
        wid = lax.axis_index("s") * NC + lax.axis_index("c")

        def gather(chunk, slot):
            src = table.at[idx_v.at[pl.ds(chunk * CHUNK, CHUNK)]]
            return pltpu.make_async_copy(src, buf_v.at[slot], sems.at[slot])

        def pool(chunk, slot):
            for b in range(CB):
                for g in range(NLG):
                    acc = buf_v[slot, b * LP, pl.ds(g * 16, 16)]
                    for j in range(1, L):
                        acc = acc + buf_v[slot, b * LP + j, pl.ds(g * 16, 16)]
                    out_v[pl.ds((chunk * CB + b) * EMBED_DIM + g * 16, 16)] = (
                        acc)

        pltpu.sync_copy(idx_hbm.at[pl.ds(wid * (RPW * LP), RPW * LP)], idx_v)
        for s in range(NBUF):
            gather(s, s).start()

        @pl.loop(0, NCH, step=NBUF)
        def _(c):
            for s in range(NBUF):
                chunk = c + s
                gather(chunk, s).wait()
                pool(chunk, s)
                nxt = chunk + NBUF

                @pl.when(nxt < NCH)
                def _():
                    gather(nxt, s).start()

        pltpu.sync_copy(
            out_v, out_hbm.at[pl.ds(wid * (RPW * EMBED_DIM),
                                    RPW * EMBED_DIM)])

    return k(table128, idx_flat)


XW = 2048
NXB = -(-VOCAB // XW)
VP = NXB * XW


def _xpose_body(t_ref, o_ref):
    o_ref[:, 0:EMBED_DIM] = t_ref[...].T


def _tc_xpose_pad(table_t):
    return pl.pallas_call(
        _xpose_body,
        grid=(NXB,),
        in_specs=[pl.BlockSpec((EMBED_DIM, XW), lambda i: (0, i))],
        out_specs=pl.BlockSpec((XW, 128), lambda i: (i, 0)),
        out_shape=jax.ShapeDtypeStruct((VP, 128), jnp.float32),
    )(table_t)


def _tc_dense_body(u_ids, c_ids, us, cs, uW, ub, cW, cb, out):
    f32 = jnp.float32
    dn = (((1,), (1,)), ((), ()))
    hi = jax.lax.Precision.HIGHEST

    cnt_u = jnp.sum((u_ids[...] != 0).astype(f32), axis=1, keepdims=True)
    pu = us[...] / jnp.maximum(cnt_u, 1.0)
    hu = jnp.tanh(lax.dot_general(pu, uW[...], dn, precision=hi,
                                  preferred_element_type=f32) + ub[...])
    hu = hu / jnp.maximum(jnp.sqrt(jnp.sum(hu * hu, 1, keepdims=True)), 1e-12)

    cnt_c = jnp.sum((c_ids[...] != 0).astype(f32), axis=1, keepdims=True)
    pc = cs[...] / jnp.maximum(cnt_c, 1.0)
    hc = jnp.tanh(lax.dot_general(pc, cW[...], dn, precision=hi,
                                  preferred_element_type=f32) + cb[...])
    hc = hc / jnp.maximum(jnp.sqrt(jnp.sum(hc * hc, 1, keepdims=True)), 1e-12)

    out[...] = jnp.sum(hu * hc, axis=1)


def _tc_dense(u_ids, c_ids, u_sum, c_sum, user_W, user_b, content_W,
              content_b):
    R = 1024
    grid = (B // R,)
    ids_spec = pl.BlockSpec((R, L), lambda i: (i, 0))
    sum_spec = pl.BlockSpec((R, EMBED_DIM), lambda i: (i, 0))
    w_spec = pl.BlockSpec((HIDDEN_DIM, EMBED_DIM), lambda i: (0, 0))
    b_spec = pl.BlockSpec((1, HIDDEN_DIM), lambda i: (0, 0))
    return pl.pallas_call(
        _tc_dense_body,
        grid=grid,
        in_specs=[ids_spec, ids_spec, sum_spec, sum_spec,
                  w_spec, b_spec, w_spec, b_spec],
        out_specs=pl.BlockSpec((R,), lambda i: (i,)),
        out_shape=jax.ShapeDtypeStruct((B,), jnp.float32),
    )(u_ids, c_ids, u_sum, c_sum, user_W, user_b.reshape(1, HIDDEN_DIM),
      content_W, content_b.reshape(1, HIDDEN_DIM))


def kernel(user_table, content_table, user_W, user_b, content_W, content_b,
           user_ids, content_ids):
    u_ids = user_ids.astype(jnp.int32)
    c_ids = content_ids.astype(jnp.int32)
    u_idx = jnp.concatenate([u_ids, u_ids[:, :LP - L]], axis=1).reshape(-1)
    c_idx = jnp.concatenate([c_ids, c_ids[:, :LP - L]], axis=1).reshape(-1)
    u_t128 = _tc_xpose_pad(user_table.T)
    u_sum = _sc_pool_tower(u_t128, u_idx)
    c_t128 = _tc_xpose_pad(content_table.T)
    c_sum = _sc_pool_tower(c_t128, c_idx)
    return _tc_dense(u_ids, c_ids, u_sum.reshape(B, EMBED_DIM),
                     c_sum.reshape(B, EMBED_DIM), user_W, user_b, content_W,
                     content_b)


# speedup vs baseline: 1.2528x; 1.2528x over previous
import functools

import jax
import jax.numpy as jnp
from jax import lax
from jax.experimental import pallas as pl
from jax.experimental.pallas import tpu as pltpu
from jax.experimental.pallas import tpu_sc as plsc

VOCAB, EMBED_DIM, HIDDEN_DIM = 1000000, 64, 128
B, L = 16384, 50
LP = 52
NC, NS = 2, 16
NW = NC * NS
RPW = B // NW
CB = 2
CHUNK = CB * LP
NCH = RPW // CB
NBUF = 4
NLG = EMBED_DIM // 16


def _sc_pool_tower(table128, idx_flat):
    mesh = plsc.VectorSubcoreMesh(core_axis_name="c", subcore_axis_name="s")

    @functools.partial(
        pl.kernel,
        out_type=jax.ShapeDtypeStruct((B * EMBED_DIM,), jnp.float32),
        mesh=mesh,
        scratch_types=[
            pltpu.VMEM((RPW * LP,), jnp.int32),
            pltpu.VMEM((NBUF, CHUNK, 128), jnp.float32),
            pltpu.VMEM((RPW * EMBED_DIM,), jnp.float32),
            pltpu.SemaphoreType.DMA((NBUF,)),
        ],
    )
    def k(table, idx_hbm, out_hbm, idx_v, buf_v, out_v, sems):
        wid = lax.axis_index("s") * NC + lax.axis_index("c")

        def gather(chunk, slot):
            src = table.at[idx_v.at[pl.ds(chunk * CHUNK, CHUNK)]]
            return pltpu.make_async_copy(src, buf_v.at[slot], sems.at[slot])

        def pool(chunk, slot):
            for b in range(CB):
                for g in range(NLG):
                    acc = buf_v[slot, b * LP, pl.ds(g * 16, 16)]
                    for j in range(1, L):
                        acc = acc + buf_v[slot, b * LP + j, pl.ds(g * 16, 16)]
                    out_v[pl.ds((chunk * CB + b) * EMBED_DIM + g * 16, 16)] = (
                        acc)

        pltpu.sync_copy(idx_hbm.at[pl.ds(wid * (RPW * LP), RPW * LP)], idx_v)
        for s in range(NBUF):
            gather(s, s).start()

        @pl.loop(0, NCH, step=NBUF)
        def _(c):
            for s in range(NBUF):
                chunk = c + s
                gather(chunk, s).wait()
                pool(chunk, s)
                nxt = chunk + NBUF

                @pl.when(nxt < NCH)
                def _():
                    gather(nxt, s).start()

        pltpu.sync_copy(
            out_v, out_hbm.at[pl.ds(wid * (RPW * EMBED_DIM),
                                    RPW * EMBED_DIM)])

    return k(table128, idx_flat)


XW = 8192
NXB = -(-VOCAB // XW)
VP = NXB * XW


def _xpose_body(t_ref, o_ref):
    o_ref[:, 0:EMBED_DIM] = t_ref[...].T


def _tc_xpose_pad(table_t):
    return pl.pallas_call(
        _xpose_body,
        grid=(NXB,),
        in_specs=[pl.BlockSpec((EMBED_DIM, XW), lambda i: (0, i))],
        out_specs=pl.BlockSpec((XW, 128), lambda i: (i, 0)),
        out_shape=jax.ShapeDtypeStruct((VP, 128), jnp.float32),
    )(table_t)


def _tc_dense_body(u_ids, c_ids, us, cs, uW, ub, cW, cb, out):
    f32 = jnp.float32
    dn = (((1,), (1,)), ((), ()))
    hi = jax.lax.Precision.HIGHEST

    cnt_u = jnp.sum((u_ids[...] != 0).astype(f32), axis=1, keepdims=True)
    pu = us[...] / jnp.maximum(cnt_u, 1.0)
    hu = jnp.tanh(lax.dot_general(pu, uW[...], dn, precision=hi,
                                  preferred_element_type=f32) + ub[...])
    hu = hu / jnp.maximum(jnp.sqrt(jnp.sum(hu * hu, 1, keepdims=True)), 1e-12)

    cnt_c = jnp.sum((c_ids[...] != 0).astype(f32), axis=1, keepdims=True)
    pc = cs[...] / jnp.maximum(cnt_c, 1.0)
    hc = jnp.tanh(lax.dot_general(pc, cW[...], dn, precision=hi,
                                  preferred_element_type=f32) + cb[...])
    hc = hc / jnp.maximum(jnp.sqrt(jnp.sum(hc * hc, 1, keepdims=True)), 1e-12)

    out[...] = jnp.sum(hu * hc, axis=1)


def _tc_dense(u_ids, c_ids, u_sum, c_sum, user_W, user_b, content_W,
              content_b):
    R = 1024
    grid = (B // R,)
    ids_spec = pl.BlockSpec((R, L), lambda i: (i, 0))
    sum_spec = pl.BlockSpec((R, EMBED_DIM), lambda i: (i, 0))
    w_spec = pl.BlockSpec((HIDDEN_DIM, EMBED_DIM), lambda i: (0, 0))
    b_spec = pl.BlockSpec((1, HIDDEN_DIM), lambda i: (0, 0))
    return pl.pallas_call(
        _tc_dense_body,
        grid=grid,
        in_specs=[ids_spec, ids_spec, sum_spec, sum_spec,
                  w_spec, b_spec, w_spec, b_spec],
        out_specs=pl.BlockSpec((R,), lambda i: (i,)),
        out_shape=jax.ShapeDtypeStruct((B,), jnp.float32),
    )(u_ids, c_ids, u_sum, c_sum, user_W, user_b.reshape(1, HIDDEN_DIM),
      content_W, content_b.reshape(1, HIDDEN_DIM))


def kernel(user_table, content_table, user_W, user_b, content_W, content_b,
           user_ids, content_ids):
    u_ids = user_ids.astype(jnp.int32)
    c_ids = content_ids.astype(jnp.int32)
    u_idx = jnp.concatenate([u_ids, u_ids[:, :LP - L]], axis=1).reshape(-1)
    c_idx = jnp.concatenate([c_ids, c_ids[:, :LP - L]], axis=1).reshape(-1)
    u_t128 = _tc_xpose_pad(user_table.T)
    u_sum = _sc_pool_tower(u_t128, u_idx)
    c_t128 = _tc_xpose_pad(content_table.T)
    c_sum = _sc_pool_tower(c_t128, c_idx)
    return _tc_dense(u_ids, c_ids, u_sum.reshape(B, EMBED_DIM),
                     c_sum.reshape(B, EMBED_DIM), user_W, user_b, content_W,
                     content_b)


# speedup vs baseline: 1.2618x; 1.0071x over previous
import functools

import jax
import jax.numpy as jnp
from jax import lax
from jax.experimental import pallas as pl
from jax.experimental.pallas import tpu as pltpu
from jax.experimental.pallas import tpu_sc as plsc

VOCAB, EMBED_DIM, HIDDEN_DIM = 1000000, 64, 128
B, L = 16384, 50
LP = 52
NC, NS = 2, 16
NW = NC * NS
RPW = B // NW
CB = 2
CHUNK = CB * LP
NCH = RPW // CB
NBUF = 4
NLG = EMBED_DIM // 16


def _sc_pool_tower(table128, idx_flat):
    mesh = plsc.VectorSubcoreMesh(core_axis_name="c", subcore_axis_name="s")

    @functools.partial(
        pl.kernel,
        out_type=jax.ShapeDtypeStruct((B * EMBED_DIM,), jnp.float32),
        mesh=mesh,
        scratch_types=[
            pltpu.VMEM((RPW * LP,), jnp.int32),
            pltpu.VMEM((NBUF, CHUNK, 128), jnp.float32),
            pltpu.VMEM((RPW * EMBED_DIM,), jnp.float32),
            pltpu.SemaphoreType.DMA((NBUF,)),
        ],
    )
    def k(table, idx_hbm, out_hbm, idx_v, buf_v, out_v, sems):
        wid = lax.axis_index("s") * NC + lax.axis_index("c")

        def gather(chunk, slot):
            src = table.at[idx_v.at[pl.ds(chunk * CHUNK, CHUNK)]]
            return pltpu.make_async_copy(src, buf_v.at[slot], sems.at[slot])

        def pool(chunk, slot):
            for b in range(CB):
                for g in range(NLG):
                    acc = buf_v[slot, b * LP, pl.ds(g * 16, 16)]
                    for j in range(1, L):
                        acc = acc + buf_v[slot, b * LP + j, pl.ds(g * 16, 16)]
                    out_v[pl.ds((chunk * CB + b) * EMBED_DIM + g * 16, 16)] = (
                        acc)

        pltpu.sync_copy(idx_hbm.at[pl.ds(wid * (RPW * LP), RPW * LP)], idx_v)
        for s in range(NBUF):
            gather(s, s).start()

        @pl.loop(0, NCH, step=NBUF)
        def _(c):
            for s in range(NBUF):
                chunk = c + s
                gather(chunk, s).wait()
                pool(chunk, s)
                nxt = chunk + NBUF

                @pl.when(nxt < NCH)
                def _():
                    gather(nxt, s).start()

        pltpu.sync_copy(
            out_v, out_hbm.at[pl.ds(wid * (RPW * EMBED_DIM),
                                    RPW * EMBED_DIM)])

    return k(table128, idx_flat)


XW = 16384
NXB = -(-VOCAB // XW)
VP = NXB * XW


def _xpose_body(t_ref, o_ref):
    o_ref[:, 0:EMBED_DIM] = t_ref[...].T


def _tc_xpose_pad(table_t):
    return pl.pallas_call(
        _xpose_body,
        grid=(NXB,),
        in_specs=[pl.BlockSpec((EMBED_DIM, XW), lambda i: (0, i))],
        out_specs=pl.BlockSpec((XW, 128), lambda i: (i, 0)),
        out_shape=jax.ShapeDtypeStruct((VP, 128), jnp.float32),
    )(table_t)


def _tc_dense_body(u_ids, c_ids, us, cs, uW, ub, cW, cb, out):
    f32 = jnp.float32
    dn = (((1,), (1,)), ((), ()))
    hi = jax.lax.Precision.HIGHEST

    cnt_u = jnp.sum((u_ids[...] != 0).astype(f32), axis=1, keepdims=True)
    pu = us[...] / jnp.maximum(cnt_u, 1.0)
    hu = jnp.tanh(lax.dot_general(pu, uW[...], dn, precision=hi,
                                  preferred_element_type=f32) + ub[...])
    hu = hu / jnp.maximum(jnp.sqrt(jnp.sum(hu * hu, 1, keepdims=True)), 1e-12)

    cnt_c = jnp.sum((c_ids[...] != 0).astype(f32), axis=1, keepdims=True)
    pc = cs[...] / jnp.maximum(cnt_c, 1.0)
    hc = jnp.tanh(lax.dot_general(pc, cW[...], dn, precision=hi,
                                  preferred_element_type=f32) + cb[...])
    hc = hc / jnp.maximum(jnp.sqrt(jnp.sum(hc * hc, 1, keepdims=True)), 1e-12)

    out[...] = jnp.sum(hu * hc, axis=1)


def _tc_dense(u_ids, c_ids, u_sum, c_sum, user_W, user_b, content_W,
              content_b):
    R = 1024
    grid = (B // R,)
    ids_spec = pl.BlockSpec((R, L), lambda i: (i, 0))
    sum_spec = pl.BlockSpec((R, EMBED_DIM), lambda i: (i, 0))
    w_spec = pl.BlockSpec((HIDDEN_DIM, EMBED_DIM), lambda i: (0, 0))
    b_spec = pl.BlockSpec((1, HIDDEN_DIM), lambda i: (0, 0))
    return pl.pallas_call(
        _tc_dense_body,
        grid=grid,
        in_specs=[ids_spec, ids_spec, sum_spec, sum_spec,
                  w_spec, b_spec, w_spec, b_spec],
        out_specs=pl.BlockSpec((R,), lambda i: (i,)),
        out_shape=jax.ShapeDtypeStruct((B,), jnp.float32),
    )(u_ids, c_ids, u_sum, c_sum, user_W, user_b.reshape(1, HIDDEN_DIM),
      content_W, content_b.reshape(1, HIDDEN_DIM))


def kernel(user_table, content_table, user_W, user_b, content_W, content_b,
           user_ids, content_ids):
    u_ids = user_ids.astype(jnp.int32)
    c_ids = content_ids.astype(jnp.int32)
    u_idx = jnp.concatenate([u_ids, u_ids[:, :LP - L]], axis=1).reshape(-1)
    c_idx = jnp.concatenate([c_ids, c_ids[:, :LP - L]], axis=1).reshape(-1)
    u_t128 = _tc_xpose_pad(user_table.T)
    u_sum = _sc_pool_tower(u_t128, u_idx)
    c_t128 = _tc_xpose_pad(content_table.T)
    c_sum = _sc_pool_tower(c_t128, c_idx)
    return _tc_dense(u_ids, c_ids, u_sum.reshape(B, EMBED_DIM),
                     c_sum.reshape(B, EMBED_DIM), user_W, user_b, content_W,
                     content_b)


# speedup vs baseline: 1.2674x; 1.0045x over previous
import functools

import jax
import jax.numpy as jnp
from jax import lax
from jax.experimental import pallas as pl
from jax.experimental.pallas import tpu as pltpu
from jax.experimental.pallas import tpu_sc as plsc

VOCAB, EMBED_DIM, HIDDEN_DIM = 1000000, 64, 128
B, L = 16384, 50
LP = 52
NC, NS = 2, 16
NW = NC * NS
RPW = B // NW
CB = 2
CHUNK = CB * LP
NCH = RPW // CB
NBUF = 4
NLG = EMBED_DIM // 16


def _sc_pool_tower(table128, idx_flat):
    mesh = plsc.VectorSubcoreMesh(core_axis_name="c", subcore_axis_name="s")

    @functools.partial(
        pl.kernel,
        out_type=jax.ShapeDtypeStruct((B * EMBED_DIM,), jnp.float32),
        mesh=mesh,
        scratch_types=[
            pltpu.VMEM((RPW * LP,), jnp.int32),
            pltpu.VMEM((NBUF, CHUNK, 128), jnp.float32),
            pltpu.VMEM((RPW * EMBED_DIM,), jnp.float32),
            pltpu.SemaphoreType.DMA((NBUF,)),
        ],
    )
    def k(table, idx_hbm, out_hbm, idx_v, buf_v, out_v, sems):
        wid = lax.axis_index("s") * NC + lax.axis_index("c")

        def gather(chunk, slot):
            src = table.at[idx_v.at[pl.ds(chunk * CHUNK, CHUNK)]]
            return pltpu.make_async_copy(src, buf_v.at[slot], sems.at[slot])

        def pool(chunk, slot):
            for b in range(CB):
                for g in range(NLG):
                    acc = buf_v[slot, b * LP, pl.ds(g * 16, 16)]
                    for j in range(1, L):
                        acc = acc + buf_v[slot, b * LP + j, pl.ds(g * 16, 16)]
                    out_v[pl.ds((chunk * CB + b) * EMBED_DIM + g * 16, 16)] = (
                        acc)

        pltpu.sync_copy(idx_hbm.at[pl.ds(wid * (RPW * LP), RPW * LP)], idx_v)
        for s in range(NBUF):
            gather(s, s).start()

        @pl.loop(0, NCH, step=NBUF)
        def _(c):
            for s in range(NBUF):
                chunk = c + s
                gather(chunk, s).wait()
                pool(chunk, s)
                nxt = chunk + NBUF

                @pl.when(nxt < NCH)
                def _():
                    gather(nxt, s).start()

        pltpu.sync_copy(
            out_v, out_hbm.at[pl.ds(wid * (RPW * EMBED_DIM),
                                    RPW * EMBED_DIM)])

    return k(table128, idx_flat)


XW = 32768
NXB = -(-VOCAB // XW)
VP = NXB * XW


def _xpose_body(t_ref, o_ref):
    o_ref[:, 0:EMBED_DIM] = t_ref[...].T


def _tc_xpose_pad(table_t):
    return pl.pallas_call(
        _xpose_body,
        grid=(NXB,),
        in_specs=[pl.BlockSpec((EMBED_DIM, XW), lambda i: (0, i))],
        out_specs=pl.BlockSpec((XW, 128), lambda i: (i, 0)),
        out_shape=jax.ShapeDtypeStruct((VP, 128), jnp.float32),
    )(table_t)


def _tc_dense_body(u_ids, c_ids, us, cs, uW, ub, cW, cb, out):
    f32 = jnp.float32
    dn = (((1,), (1,)), ((), ()))
    hi = jax.lax.Precision.HIGHEST

    cnt_u = jnp.sum((u_ids[...] != 0).astype(f32), axis=1, keepdims=True)
    pu = us[...] / jnp.maximum(cnt_u, 1.0)
    hu = jnp.tanh(lax.dot_general(pu, uW[...], dn, precision=hi,
                                  preferred_element_type=f32) + ub[...])
    hu = hu / jnp.maximum(jnp.sqrt(jnp.sum(hu * hu, 1, keepdims=True)), 1e-12)

    cnt_c = jnp.sum((c_ids[...] != 0).astype(f32), axis=1, keepdims=True)
    pc = cs[...] / jnp.maximum(cnt_c, 1.0)
    hc = jnp.tanh(lax.dot_general(pc, cW[...], dn, precision=hi,
                                  preferred_element_type=f32) + cb[...])
    hc = hc / jnp.maximum(jnp.sqrt(jnp.sum(hc * hc, 1, keepdims=True)), 1e-12)

    out[...] = jnp.sum(hu * hc, axis=1)


def _tc_dense(u_ids, c_ids, u_sum, c_sum, user_W, user_b, content_W,
              content_b):
    R = 1024
    grid = (B // R,)
    ids_spec = pl.BlockSpec((R, L), lambda i: (i, 0))
    sum_spec = pl.BlockSpec((R, EMBED_DIM), lambda i: (i, 0))
    w_spec = pl.BlockSpec((HIDDEN_DIM, EMBED_DIM), lambda i: (0, 0))
    b_spec = pl.BlockSpec((1, HIDDEN_DIM), lambda i: (0, 0))
    return pl.pallas_call(
        _tc_dense_body,
        grid=grid,
        in_specs=[ids_spec, ids_spec, sum_spec, sum_spec,
                  w_spec, b_spec, w_spec, b_spec],
        out_specs=pl.BlockSpec((R,), lambda i: (i,)),
        out_shape=jax.ShapeDtypeStruct((B,), jnp.float32),
    )(u_ids, c_ids, u_sum, c_sum, user_W, user_b.reshape(1, HIDDEN_DIM),
      content_W, content_b.reshape(1, HIDDEN_DIM))


def kernel(user_table, content_table, user_W, user_b, content_W, content_b,
           user_ids, content_ids):
    u_ids = user_ids.astype(jnp.int32)
    c_ids = content_ids.astype(jnp.int32)
    u_idx = jnp.concatenate([u_ids, u_ids[:, :LP - L]], axis=1).reshape(-1)
    c_idx = jnp.concatenate([c_ids, c_ids[:, :LP - L]], axis=1).reshape(-1)
    u_t128 = _tc_xpose_pad(user_table.T)
    u_sum = _sc_pool_tower(u_t128, u_idx)
    c_t128 = _tc_xpose_pad(content_table.T)
    c_sum = _sc_pool_tower(c_t128, c_idx)
    return _tc_dense(u_ids, c_ids, u_sum.reshape(B, EMBED_DIM),
                     c_sum.reshape(B, EMBED_DIM), user_W, user_b, content_W,
                     content_b)


# speedup vs baseline: 1.2813x; 1.0110x over previous
import functools

import jax
import jax.numpy as jnp
from jax import lax
from jax.experimental import pallas as pl
from jax.experimental.pallas import tpu as pltpu
from jax.experimental.pallas import tpu_sc as plsc

VOCAB, EMBED_DIM, HIDDEN_DIM = 1000000, 64, 128
B, L = 16384, 50
LP = 52
NC, NS = 2, 16
NW = NC * NS
RPW = B // NW
CB = 2
CHUNK = CB * LP
NCH = RPW // CB
NBUF = 4
NLG = EMBED_DIM // 16


def _sc_pool_tower(table128, idx_flat):
    mesh = plsc.VectorSubcoreMesh(core_axis_name="c", subcore_axis_name="s")

    @functools.partial(
        pl.kernel,
        out_type=jax.ShapeDtypeStruct((B * EMBED_DIM,), jnp.float32),
        mesh=mesh,
        scratch_types=[
            pltpu.VMEM((RPW * LP,), jnp.int32),
            pltpu.VMEM((NBUF, CHUNK, 128), jnp.float32),
            pltpu.VMEM((RPW * EMBED_DIM,), jnp.float32),
            pltpu.SemaphoreType.DMA((NBUF,)),
        ],
    )
    def k(table, idx_hbm, out_hbm, idx_v, buf_v, out_v, sems):
        wid = lax.axis_index("s") * NC + lax.axis_index("c")

        def gather(chunk, slot):
            src = table.at[idx_v.at[pl.ds(chunk * CHUNK, CHUNK)]]
            return pltpu.make_async_copy(src, buf_v.at[slot], sems.at[slot])

        def pool(chunk, slot):
            for b in range(CB):
                for g in range(NLG):
                    acc = buf_v[slot, b * LP, pl.ds(g * 16, 16)]
                    for j in range(1, L):
                        acc = acc + buf_v[slot, b * LP + j, pl.ds(g * 16, 16)]
                    out_v[pl.ds((chunk * CB + b) * EMBED_DIM + g * 16, 16)] = (
                        acc)

        pltpu.sync_copy(idx_hbm.at[pl.ds(wid * (RPW * LP), RPW * LP)], idx_v)
        for s in range(NBUF):
            gather(s, s).start()

        @pl.loop(0, NCH, step=NBUF)
        def _(c):
            for s in range(NBUF):
                chunk = c + s
                gather(chunk, s).wait()
                pool(chunk, s)
                nxt = chunk + NBUF

                @pl.when(nxt < NCH)
                def _():
                    gather(nxt, s).start()

        pltpu.sync_copy(
            out_v, out_hbm.at[pl.ds(wid * (RPW * EMBED_DIM),
                                    RPW * EMBED_DIM)])

    return k(table128, idx_flat)


XW = 32768
NXB = -(-VOCAB // XW)
VP = NXB * XW


def _xpose_body(t_ref, o_ref):
    o_ref[:, 0:EMBED_DIM] = t_ref[...].T


def _tc_xpose_pad(table_t):
    return pl.pallas_call(
        _xpose_body,
        grid=(NXB,),
        in_specs=[pl.BlockSpec((EMBED_DIM, XW), lambda i: (0, i))],
        out_specs=pl.BlockSpec((XW, 128), lambda i: (i, 0)),
        out_shape=jax.ShapeDtypeStruct((VP, 128), jnp.float32),
    )(table_t)


def _tower_vec(ids, s, W, bias):
    f32 = jnp.float32
    cnt = jnp.sum((ids != 0).astype(f32), axis=1, keepdims=True)
    p = s / jnp.maximum(cnt, 1.0)
    h = jnp.tanh(lax.dot_general(p, W, (((1,), (1,)), ((), ())),
                                 precision=jax.lax.Precision.HIGHEST,
                                 preferred_element_type=f32) + bias)
    return h / jnp.maximum(jnp.sqrt(jnp.sum(h * h, 1, keepdims=True)), 1e-12)


def _tc_u_body(ids_ref, s_ref, w_ref, b_ref, h_ref):
    h_ref[...] = _tower_vec(ids_ref[...], s_ref[...], w_ref[...], b_ref[...])


def _tc_c_body(ids_ref, s_ref, w_ref, b_ref, hu_ref, out_ref):
    hc = _tower_vec(ids_ref[...], s_ref[...], w_ref[...], b_ref[...])
    out_ref[...] = jnp.sum(hu_ref[...] * hc, axis=1)


_R = 1024
_ids_spec = pl.BlockSpec((_R, L), lambda i: (i, 0))
_sum_spec = pl.BlockSpec((_R, EMBED_DIM), lambda i: (i, 0))
_h_spec = pl.BlockSpec((_R, HIDDEN_DIM), lambda i: (i, 0))
_w_spec = pl.BlockSpec((HIDDEN_DIM, EMBED_DIM), lambda i: (0, 0))
_b_spec = pl.BlockSpec((1, HIDDEN_DIM), lambda i: (0, 0))


def _tc_u_vec(u_ids, u_sum, user_W, user_b):
    return pl.pallas_call(
        _tc_u_body,
        grid=(B // _R,),
        in_specs=[_ids_spec, _sum_spec, _w_spec, _b_spec],
        out_specs=_h_spec,
        out_shape=jax.ShapeDtypeStruct((B, HIDDEN_DIM), jnp.float32),
    )(u_ids, u_sum, user_W, user_b.reshape(1, HIDDEN_DIM))


def _tc_c_dot(c_ids, c_sum, content_W, content_b, hu):
    return pl.pallas_call(
        _tc_c_body,
        grid=(B // _R,),
        in_specs=[_ids_spec, _sum_spec, _w_spec, _b_spec, _h_spec],
        out_specs=pl.BlockSpec((_R,), lambda i: (i,)),
        out_shape=jax.ShapeDtypeStruct((B,), jnp.float32),
    )(c_ids, c_sum, content_W, content_b.reshape(1, HIDDEN_DIM), hu)


def kernel(user_table, content_table, user_W, user_b, content_W, content_b,
           user_ids, content_ids):
    u_ids = user_ids.astype(jnp.int32)
    c_ids = content_ids.astype(jnp.int32)
    u_idx = jnp.concatenate([u_ids, u_ids[:, :LP - L]], axis=1).reshape(-1)
    c_idx = jnp.concatenate([c_ids, c_ids[:, :LP - L]], axis=1).reshape(-1)
    u_t128 = _tc_xpose_pad(user_table.T)
    u_sum = _sc_pool_tower(u_t128, u_idx)
    c_t128 = _tc_xpose_pad(content_table.T)
    c_sum = _sc_pool_tower(c_t128, c_idx)
    hu = _tc_u_vec(u_ids, u_sum.reshape(B, EMBED_DIM), user_W, user_b)
    return _tc_c_dot(c_ids, c_sum.reshape(B, EMBED_DIM), content_W,
                     content_b, hu)
